# baseline (speedup 1.0000x reference)
import functools

import jax
import jax.numpy as jnp
from jax import lax
from jax.experimental import pallas as pl
from jax.experimental.pallas import tpu as pltpu
from jax.experimental.pallas import tpu_sc as plsc

EMB = 64
B = 4096
T = 200
NC = 2
NS = 16
NW = NC * NS
TOTAL = B * T
PER_W = TOTAL // NW
CHUNK = 128
NCHUNK = PER_W // CHUNK

_mesh = plsc.VectorSubcoreMesh(core_axis_name="c", subcore_axis_name="s")


@functools.partial(
    pl.kernel,
    out_type=jax.ShapeDtypeStruct((TOTAL, EMB), jnp.float32),
    mesh=_mesh,
    scratch_types=[
        pltpu.VMEM((NCHUNK, CHUNK), jnp.int32),
        pltpu.VMEM((CHUNK, EMB), jnp.float32),
        pltpu.SemaphoreType.DMA,
    ],
    compiler_params=pltpu.CompilerParams(use_tc_tiling_on_sc=False),
)
def _gather(table_hbm, toks_hbm, out_hbm, idx_v, rows_v, sem):
    wid = lax.axis_index("s") * NC + lax.axis_index("c")
    base = wid * PER_W
    pltpu.sync_copy(toks_hbm.at[wid], idx_v)

    @pl.loop(0, NCHUNK)
    def _(j):
        pltpu.async_copy(table_hbm.at[idx_v.at[j]], rows_v, sem).wait()
        pltpu.sync_copy(rows_v, out_hbm.at[pl.ds(base + j * CHUNK, CHUNK)])


def kernel(toks, table):
    idx = toks.reshape(NW, NCHUNK, CHUNK)
    out = _gather(table, idx)
    return out.reshape(B, T, EMB)


# speedup vs baseline: 1.1186x; 1.1186x over previous
import functools

import jax
import jax.numpy as jnp
from jax import lax
from jax.experimental import pallas as pl
from jax.experimental.pallas import tpu as pltpu
from jax.experimental.pallas import tpu_sc as plsc

EMB = 64
B = 4096
T = 200
NC = 2
NS = 16
NW = NC * NS
TOTAL = B * T
PER_W = TOTAL // NW
CHUNK = 128
KS = 4
GROUP = CHUNK * KS
NGROUP = PER_W // GROUP
NCHUNK = PER_W // CHUNK

_mesh = plsc.VectorSubcoreMesh(core_axis_name="c", subcore_axis_name="s")


@functools.partial(
    pl.kernel,
    out_type=jax.ShapeDtypeStruct((TOTAL, EMB), jnp.float32),
    mesh=_mesh,
    scratch_types=[
        pltpu.VMEM((NCHUNK, CHUNK), jnp.int32),
        pltpu.VMEM((2, GROUP, EMB), jnp.float32),
        pltpu.SemaphoreType.DMA,
        pltpu.SemaphoreType.DMA,
    ],
    compiler_params=pltpu.CompilerParams(use_tc_tiling_on_sc=False),
)
def _gather(table_hbm, toks_hbm, out_hbm, idx_v, rows_v, gsem, psem):
    wid = lax.axis_index("s") * NC + lax.axis_index("c")
    base = wid * PER_W

    def start_gathers(g, b):
        for k in range(KS):
            pltpu.async_copy(
                table_hbm.at[idx_v.at[g * KS + k]],
                rows_v.at[b].at[pl.ds(k * CHUNK, CHUNK)],
                gsem)

    def wait_gathers(g, b):
        for k in range(KS):
            pltpu.make_async_copy(
                table_hbm.at[idx_v.at[g * KS + k]],
                rows_v.at[b].at[pl.ds(k * CHUNK, CHUNK)],
                gsem).wait()

    def start_put(g, b):
        pltpu.async_copy(
            rows_v.at[b], out_hbm.at[pl.ds(base + g * GROUP, GROUP)], psem)

    def wait_put():
        pltpu.make_async_copy(
            rows_v.at[0], out_hbm.at[pl.ds(base, GROUP)], psem).wait()

    pltpu.sync_copy(toks_hbm.at[wid], idx_v)

    start_gathers(0, 0)
    start_gathers(1, 1)
    wait_gathers(0, 0)
    start_put(0, 0)

    @pl.loop(1, NGROUP - 1)
    def _(g):
        b = lax.rem(g, 2)
        nb = 1 - b
        wait_put()
        start_gathers(g + 1, nb)
        wait_gathers(g, b)
        start_put(g, b)

    bl = (NGROUP - 1) % 2
    wait_put()
    wait_gathers(NGROUP - 1, bl)
    start_put(NGROUP - 1, bl)
    wait_put()


def kernel(toks, table):
    idx = toks.reshape(NW, NCHUNK, CHUNK)
    out = _gather(table, idx)
    return out.reshape(B, T, EMB)
